# baseline (speedup 1.0000x reference)
import jax
import jax.numpy as jnp
from jax.experimental import pallas as pl
from jax.experimental.pallas import tpu as pltpu

B, T, L = 8, 32, 64
D = 768
H = 12
dh = D // H
G = 4
NG = H // G
TT = 16
NT = T // TT


def _kv_body(edu_ref, wkt_ref, wvt_ref, k4_ref, v4_ref):
    f32 = jnp.float32
    bf16 = jnp.bfloat16
    edu16 = edu_ref[0].astype(bf16)
    kT = jax.lax.dot_general(wkt_ref[:], edu16, (((0,), (1,)), ((), ())),
                             preferred_element_type=f32).astype(bf16)
    v16 = jax.lax.dot(edu16, wvt_ref[:],
                      preferred_element_type=f32).astype(bf16)
    k4_ref[0] = jnp.zeros((D, G * T), bf16)
    v4_ref[0] = jnp.zeros((NG * G * T, G * dh), bf16)
    for h in range(H):
        j, r = divmod(h, G)
        k4_ref[0, G * dh * j + dh * r:G * dh * j + dh * (r + 1),
               T * r:T * (r + 1)] = kT[dh * h:dh * (h + 1), :]
        v4_ref[0, G * T * j + T * r:G * T * j + T * (r + 1),
               dh * r:dh * (r + 1)] = v16[:, dh * h:dh * (h + 1)]


def _main_body(spk4_ref, spk_col_ref, tok_ref, k4_ref, v4_ref, wqt_ref,
               out_ref):
    f32 = jnp.float32
    bf16 = jnp.bfloat16
    i = pl.program_id(1)
    t0 = i * TT

    tok = tok_ref[0]
    q = jax.lax.dot(tok.astype(bf16), wqt_ref[:],
                    preferred_element_type=f32).astype(bf16)

    spk4 = spk4_ref[0]
    spk_col = spk_col_ref[0]
    same = spk_col == spk4
    trow = jax.lax.broadcasted_iota(jnp.int32, (TT, G * T), 0) + t0
    ucol = jax.lax.broadcasted_iota(jnp.int32, (TT, G * T), 1) & (T - 1)
    mask4 = (same & (ucol <= trow)).astype(f32).reshape(TT, 1, G * T)

    for j in range(NG):
        csl = slice(G * dh * j, G * dh * (j + 1))
        s = jax.lax.dot(q[:, csl], k4_ref[0, csl, :],
                        preferred_element_type=f32)
        s = (s.reshape(TT, L, G * T) * mask4).astype(bf16)
        a_j = jax.lax.dot(s.reshape(TT * L, G * T),
                          v4_ref[0, G * T * j:G * T * (j + 1), :],
                          preferred_element_type=f32)
        out_ref[0, :, csl] = tok[:, csl] + a_j


def kernel(input_ids, speaker_names, token_embeddings, edu_embeddings,
           Wk, Wv, Wq):
    bf16 = jnp.bfloat16
    tok = token_embeddings.reshape(B, T * L, D)
    spk = speaker_names.astype(jnp.int32)
    spk4 = jnp.tile(spk.reshape(B, 1, T), (1, 1, G))
    spk_col = spk.reshape(B, T, 1)

    k4, v4 = pl.pallas_call(
        _kv_body,
        grid=(B,),
        in_specs=[
            pl.BlockSpec((1, T, D), lambda b: (b, 0, 0)),
            pl.BlockSpec((D, D), lambda b: (0, 0)),
            pl.BlockSpec((D, D), lambda b: (0, 0)),
        ],
        out_specs=[
            pl.BlockSpec((1, D, G * T), lambda b: (b, 0, 0)),
            pl.BlockSpec((1, NG * G * T, G * dh), lambda b: (b, 0, 0)),
        ],
        out_shape=[
            jax.ShapeDtypeStruct((B, D, G * T), bf16),
            jax.ShapeDtypeStruct((B, NG * G * T, G * dh), bf16),
        ],
    )(edu_embeddings, Wk.T.astype(bf16), Wv.T.astype(bf16))

    out = pl.pallas_call(
        _main_body,
        grid=(B, NT),
        in_specs=[
            pl.BlockSpec((1, 1, G * T), lambda b, i: (b, 0, 0)),
            pl.BlockSpec((1, TT, 1), lambda b, i: (b, i, 0)),
            pl.BlockSpec((1, TT * L, D), lambda b, i: (b, i, 0)),
            pl.BlockSpec((1, D, G * T), lambda b, i: (b, 0, 0)),
            pl.BlockSpec((1, NG * G * T, G * dh), lambda b, i: (b, 0, 0)),
            pl.BlockSpec((D, D), lambda b, i: (0, 0)),
        ],
        out_specs=pl.BlockSpec((1, TT * L, D), lambda b, i: (b, i, 0)),
        out_shape=jax.ShapeDtypeStruct((B, T * L, D), jnp.float32),
        compiler_params=pltpu.CompilerParams(
            dimension_semantics=("parallel", "parallel")),
    )(spk4, spk_col, tok, k4, v4, Wq.T.astype(bf16))
    return out.reshape(B, T, L, D)


# speedup vs baseline: 1.0277x; 1.0277x over previous
import jax
import jax.numpy as jnp
from jax.experimental import pallas as pl
from jax.experimental.pallas import tpu as pltpu

B, T, L = 8, 32, 64
D = 768
H = 12
dh = D // H
G = 4
NG = H // G
GT = G * T
TT = 16
NT = T // TT


def _kv_body(edu_ref, wkt_ref, wvt_ref, wqt_ref, w2_ref, v4_ref, k4_scr):
    f32 = jnp.float32
    bf16 = jnp.bfloat16
    edu16 = edu_ref[0].astype(bf16)
    kT = jax.lax.dot_general(wkt_ref[:], edu16, (((0,), (1,)), ((), ())),
                             preferred_element_type=f32).astype(bf16)
    v16 = jax.lax.dot(edu16, wvt_ref[:],
                      preferred_element_type=f32).astype(bf16)
    k4_scr[:] = jnp.zeros((D, GT), bf16)
    v4_ref[0] = jnp.zeros((NG * GT, G * dh), bf16)
    for h in range(H):
        j, r = divmod(h, G)
        k4_scr[G * dh * j + dh * r:G * dh * j + dh * (r + 1),
               T * r:T * (r + 1)] = kT[dh * h:dh * (h + 1), :]
        v4_ref[0, GT * j + T * r:GT * j + T * (r + 1),
               dh * r:dh * (r + 1)] = v16[:, dh * h:dh * (h + 1)]
    for j in range(NG):
        csl = slice(G * dh * j, G * dh * (j + 1))
        w2_ref[0, :, GT * j:GT * (j + 1)] = jax.lax.dot(
            wqt_ref[:, csl], k4_scr[csl, :],
            preferred_element_type=f32).astype(bf16)


def _main_body(spk12_ref, spk_col_ref, tok_ref, w2_ref, v4_ref, out_ref):
    f32 = jnp.float32
    bf16 = jnp.bfloat16
    i = pl.program_id(1)
    t0 = i * TT

    tok = tok_ref[0]
    s_all = jax.lax.dot(tok.astype(bf16), w2_ref[0],
                        preferred_element_type=f32)

    spk12 = spk12_ref[0]
    spk_col = spk_col_ref[0]
    same = spk_col == spk12
    trow = jax.lax.broadcasted_iota(jnp.int32, (TT, NG * GT), 0) + t0
    ucol = jax.lax.broadcasted_iota(jnp.int32, (TT, NG * GT), 1) & (T - 1)
    mask12 = (same & (ucol <= trow)).astype(f32).reshape(TT, 1, NG * GT)

    s16 = (s_all.reshape(TT, L, NG * GT) * mask12).astype(bf16)
    s16 = s16.reshape(TT * L, NG * GT)
    for j in range(NG):
        csl = slice(G * dh * j, G * dh * (j + 1))
        a_j = jax.lax.dot(s16[:, GT * j:GT * (j + 1)],
                          v4_ref[0, GT * j:GT * (j + 1), :],
                          preferred_element_type=f32)
        out_ref[0, :, csl] = tok[:, csl] + a_j


def kernel(input_ids, speaker_names, token_embeddings, edu_embeddings,
           Wk, Wv, Wq):
    bf16 = jnp.bfloat16
    tok = token_embeddings.reshape(B, T * L, D)
    spk = speaker_names.astype(jnp.int32)
    spk12 = jnp.tile(spk.reshape(B, 1, T), (1, 1, H))
    spk_col = spk.reshape(B, T, 1)

    w2, v4 = pl.pallas_call(
        _kv_body,
        grid=(B,),
        in_specs=[
            pl.BlockSpec((1, T, D), lambda b: (b, 0, 0)),
            pl.BlockSpec((D, D), lambda b: (0, 0)),
            pl.BlockSpec((D, D), lambda b: (0, 0)),
            pl.BlockSpec((D, D), lambda b: (0, 0)),
        ],
        out_specs=[
            pl.BlockSpec((1, D, NG * GT), lambda b: (b, 0, 0)),
            pl.BlockSpec((1, NG * GT, G * dh), lambda b: (b, 0, 0)),
        ],
        out_shape=[
            jax.ShapeDtypeStruct((B, D, NG * GT), bf16),
            jax.ShapeDtypeStruct((B, NG * GT, G * dh), bf16),
        ],
        scratch_shapes=[pltpu.VMEM((D, GT), bf16)],
    )(edu_embeddings, Wk.T.astype(bf16), Wv.T.astype(bf16),
      Wq.T.astype(bf16))

    out = pl.pallas_call(
        _main_body,
        grid=(B, NT),
        in_specs=[
            pl.BlockSpec((1, 1, NG * GT), lambda b, i: (b, 0, 0)),
            pl.BlockSpec((1, TT, 1), lambda b, i: (b, i, 0)),
            pl.BlockSpec((1, TT * L, D), lambda b, i: (b, i, 0)),
            pl.BlockSpec((1, D, NG * GT), lambda b, i: (b, 0, 0)),
            pl.BlockSpec((1, NG * GT, G * dh), lambda b, i: (b, 0, 0)),
        ],
        out_specs=pl.BlockSpec((1, TT * L, D), lambda b, i: (b, i, 0)),
        out_shape=jax.ShapeDtypeStruct((B, T * L, D), jnp.float32),
        compiler_params=pltpu.CompilerParams(
            dimension_semantics=("parallel", "parallel")),
    )(spk12, spk_col, tok, w2, v4)
    return out.reshape(B, T, L, D)


# speedup vs baseline: 1.0960x; 1.0664x over previous
import jax
import jax.numpy as jnp
from jax.experimental import pallas as pl
from jax.experimental.pallas import tpu as pltpu

B, T, L = 8, 32, 64
D = 768
H = 12
dh = D // H
G = 4
NG = H // G
GT = G * T
TT = 16
NT = T // TT


def _body(spk12_ref, spk_col_ref, tok_ref, edu_ref, wkt_ref, wvt_ref,
          wqt_ref, out_ref, w2_scr, v4_scr, k4_scr):
    f32 = jnp.float32
    bf16 = jnp.bfloat16
    b = pl.program_id(0)
    i = pl.program_id(1)
    t0 = i * TT

    @pl.when((b == 0) & (i == 0))
    def _():
        k4_scr[:] = jnp.zeros((D, GT), bf16)
        v4_scr[:] = jnp.zeros((NG * GT, G * dh), bf16)

    @pl.when(i == 0)
    def _():
        edu16 = edu_ref[0].astype(bf16)
        kT = jax.lax.dot_general(wkt_ref[:], edu16, (((0,), (1,)), ((), ())),
                                 preferred_element_type=f32).astype(bf16)
        v16 = jax.lax.dot(edu16, wvt_ref[:],
                          preferred_element_type=f32).astype(bf16)
        for h in range(H):
            j, r = divmod(h, G)
            k4_scr[G * dh * j + dh * r:G * dh * j + dh * (r + 1),
                   T * r:T * (r + 1)] = kT[dh * h:dh * (h + 1), :]
            v4_scr[GT * j + T * r:GT * j + T * (r + 1),
                   dh * r:dh * (r + 1)] = v16[:, dh * h:dh * (h + 1)]
        for j in range(NG):
            csl = slice(G * dh * j, G * dh * (j + 1))
            w2_scr[:, GT * j:GT * (j + 1)] = jax.lax.dot(
                wqt_ref[:, csl], k4_scr[csl, :],
                preferred_element_type=f32).astype(bf16)

    tok = tok_ref[0]
    s_all = jax.lax.dot(tok.astype(bf16), w2_scr[:],
                        preferred_element_type=f32)

    spk12 = spk12_ref[0]
    spk_col = spk_col_ref[0]
    same = spk_col == spk12
    trow = jax.lax.broadcasted_iota(jnp.int32, (TT, NG * GT), 0) + t0
    ucol = jax.lax.broadcasted_iota(jnp.int32, (TT, NG * GT), 1) & (T - 1)
    mask12 = (same & (ucol <= trow)).astype(f32).reshape(TT, 1, NG * GT)

    s16 = (s_all.reshape(TT, L, NG * GT) * mask12).astype(bf16)
    s16 = s16.reshape(TT * L, NG * GT)
    for j in range(NG):
        csl = slice(G * dh * j, G * dh * (j + 1))
        a_j = jax.lax.dot(s16[:, GT * j:GT * (j + 1)],
                          v4_scr[GT * j:GT * (j + 1), :],
                          preferred_element_type=f32)
        out_ref[0, :, csl] = tok[:, csl] + a_j


def kernel(input_ids, speaker_names, token_embeddings, edu_embeddings,
           Wk, Wv, Wq):
    bf16 = jnp.bfloat16
    tok = token_embeddings.reshape(B, T * L, D)
    spk = speaker_names.astype(jnp.int32)
    spk12 = jnp.tile(spk.reshape(B, 1, T), (1, 1, H))
    spk_col = spk.reshape(B, T, 1)

    out = pl.pallas_call(
        _body,
        grid=(B, NT),
        in_specs=[
            pl.BlockSpec((1, 1, NG * GT), lambda b, i: (b, 0, 0)),
            pl.BlockSpec((1, TT, 1), lambda b, i: (b, i, 0)),
            pl.BlockSpec((1, TT * L, D), lambda b, i: (b, i, 0)),
            pl.BlockSpec((1, T, D), lambda b, i: (b, 0, 0)),
            pl.BlockSpec((D, D), lambda b, i: (0, 0)),
            pl.BlockSpec((D, D), lambda b, i: (0, 0)),
            pl.BlockSpec((D, D), lambda b, i: (0, 0)),
        ],
        out_specs=pl.BlockSpec((1, TT * L, D), lambda b, i: (b, i, 0)),
        out_shape=jax.ShapeDtypeStruct((B, T * L, D), jnp.float32),
        scratch_shapes=[
            pltpu.VMEM((D, NG * GT), jnp.bfloat16),
            pltpu.VMEM((NG * GT, G * dh), jnp.bfloat16),
            pltpu.VMEM((D, GT), jnp.bfloat16),
        ],
    )(spk12, spk_col, tok, edu_embeddings, Wk.T.astype(bf16),
      Wv.T.astype(bf16), Wq.T.astype(bf16))
    return out.reshape(B, T, L, D)
